# baseline (speedup 1.0000x reference)
import functools

import jax
import jax.numpy as jnp
from jax import lax
from jax.experimental import pallas as pl
from jax.experimental.pallas import tpu as pltpu
from jax.experimental.pallas import tpu_sc as plsc

N = 10000
E = 320000
D_IN = 128
H = 8
D = 16
ROW = H * D
AROW = ROW + 16

NC = 2
NS = 16
NW = NC * NS
EPW = E // NW
CHUNK = 40
NCHUNK = EPW // CHUNK
NPAD = 10240
RPT = NPAD // NS
WCH = 40
SCBYTES = CHUNK * AROW * 4
assert EPW % CHUNK == 0 and NCHUNK % 2 == 0 and RPT % WCH == 0


def _mm_body(f_ref, w_ref, o_ref):
    o_ref[...] = jnp.dot(f_ref[...], w_ref[...],
                         preferred_element_type=jnp.float32) * 0.5


def _matmul(feat, W):
    blk = 2000
    return pl.pallas_call(
        _mm_body,
        grid=(N // blk,),
        in_specs=[pl.BlockSpec((blk, D_IN), lambda i: (i, 0)),
                  pl.BlockSpec((D_IN, ROW), lambda i: (0, 0))],
        out_specs=pl.BlockSpec((blk, ROW), lambda i: (i, 0)),
        out_shape=jax.ShapeDtypeStruct((N, ROW), jnp.float32),
    )(feat, W)


def _edge_kernel_body(ft_hbm, ei_hbm, out_hbm,
                      ids0, ids1, sdst0, sdst1,
                      rows0, rows1, arows0, arows1,
                      acc_sh, sem_g0, sem_g1, sem_s0, sem_s1,
                      sem_i0, sem_i1):
    cid = lax.axis_index("c")
    sid = lax.axis_index("s")
    wid = cid * NS + sid
    ebase = wid * EPW

    ids = (ids0, ids1)
    sdst = (sdst0, sdst1)
    rows = (rows0, rows1)
    arows = (arows0, arows1)
    sem_g = (sem_g0, sem_g1)
    sem_s = (sem_s0, sem_s1)
    sem_i = (sem_i0, sem_i1)

    for r in range(WCH):
        for c in range(AROW // 16):
            arows0[r, pl.ds(16 * c, 16)] = jnp.zeros((16,), jnp.float32)

    def _zero_body(i, _):
        pltpu.sync_copy(arows0, acc_sh.at[pl.ds(sid * RPT + i * WCH, WCH)])
        return 0
    lax.fori_loop(0, RPT // WCH, _zero_body, 0)
    plsc.subcore_barrier()

    one_hot = [(lax.iota(jnp.int32, 16) == h).astype(jnp.float32)
               for h in range(H)]
    dst_lanes = [(lax.iota(jnp.int32, 16) + k0) * 2 + 1
                 for k0 in (0, 16, CHUNK - 16)]

    pltpu.sync_copy(ei_hbm.at[pl.ds(2 * ebase, 2 * CHUNK)], ids0)
    pltpu.async_copy(ft_hbm.at[ids0], rows0, sem_g0)
    pltpu.async_copy(ei_hbm.at[pl.ds(2 * (ebase + CHUNK), 2 * CHUNK)],
                     ids1, sem_i1)

    def _half_body(j, b):
        o = 1 - b
        pltpu.make_async_copy(ei_hbm.at[pl.ds(0, 2 * CHUNK)],
                              ids[o], sem_i[o]).wait()
        pltpu.async_copy(ft_hbm.at[ids[o]], rows[o], sem_g[o])
        pltpu.make_async_copy(ft_hbm.at[ids[b]], rows[b], sem_g[b]).wait()

        @pl.when(j >= 2)
        def _wait_prev_scatter():
            pltpu.make_async_copy(
                arows[b], acc_sh.at[sdst[b]], sem_s[b]).wait()

        for i, k in enumerate((0, 16, CHUNK - 16)):
            sdst[b][pl.ds(k, 16)] = plsc.load_gather(ids[b], [dst_lanes[i]])
        nbase = ebase + jnp.minimum(j + 2, NCHUNK - 1) * CHUNK
        pltpu.async_copy(ei_hbm.at[pl.ds(2 * nbase, 2 * CHUNK)], ids[b],
                         sem_i[b])

        @plsc.parallel_loop(0, CHUNK, 1, unroll=2)
        def _edge_body(e):
            sv = [rows[b][2 * e, pl.ds(16 * h, 16)] for h in range(H)]
            dv = [rows[b][2 * e + 1, pl.ds(16 * h, 16)] for h in range(H)]
            cs = [plsc.cumsum(sv[h] * dv[h]) for h in range(H)]
            m = [one_hot[h] * cs[h][15] for h in range(H)]
            a_vec = ((m[0] + m[1]) + (m[2] + m[3])) + \
                    ((m[4] + m[5]) + (m[6] + m[7]))
            ex = jnp.exp(a_vec)
            arows[b][e, pl.ds(ROW, 16)] = ex
            for h in range(H):
                arows[b][e, pl.ds(16 * h, 16)] = sv[h] * ex[h]

        pltpu.async_copy(arows[b], acc_sh.at[sdst[b]], sem_s[b], add=True)

    def _pair_body(jj, _):
        _half_body(jj * 2, 0)
        _half_body(jj * 2 + 1, 1)
        return 0
    lax.fori_loop(0, NCHUNK // 2, _pair_body, 0)

    pltpu.make_async_copy(arows0, acc_sh.at[sdst0], sem_s0).wait()
    pltpu.make_async_copy(arows1, acc_sh.at[sdst1], sem_s1).wait()
    pltpu.make_async_copy(ft_hbm.at[ids0], rows0, sem_g0).wait()
    pltpu.make_async_copy(ei_hbm.at[pl.ds(0, 2 * CHUNK)],
                          ids1, sem_i1).wait()
    plsc.subcore_barrier()

    def _out_body(i, _):
        rbase = sid * RPT + i * WCH
        pltpu.sync_copy(acc_sh.at[pl.ds(rbase, WCH)], arows0)
        pltpu.sync_copy(arows0, out_hbm.at[cid, pl.ds(rbase, WCH)])
        return 0
    lax.fori_loop(0, RPT // WCH, _out_body, 0)


def _edge_pass(ft, edge_index):
    mesh = plsc.VectorSubcoreMesh(core_axis_name="c", subcore_axis_name="s")
    k = functools.partial(
        pl.kernel,
        mesh=mesh,
        out_type=jax.ShapeDtypeStruct((NC, NPAD, AROW), jnp.float32),
        scratch_types=[
            pltpu.VMEM((2 * CHUNK,), jnp.int32),
            pltpu.VMEM((2 * CHUNK,), jnp.int32),
            pltpu.VMEM((CHUNK,), jnp.int32),
            pltpu.VMEM((CHUNK,), jnp.int32),
            pltpu.VMEM((2 * CHUNK, ROW), jnp.float32),
            pltpu.VMEM((2 * CHUNK, ROW), jnp.float32),
            pltpu.VMEM((CHUNK, AROW), jnp.float32),
            pltpu.VMEM((CHUNK, AROW), jnp.float32),
            pltpu.VMEM_SHARED((NPAD, AROW), jnp.float32),
            pltpu.SemaphoreType.DMA,
            pltpu.SemaphoreType.DMA,
            pltpu.SemaphoreType.DMA,
            pltpu.SemaphoreType.DMA,
            pltpu.SemaphoreType.DMA,
            pltpu.SemaphoreType.DMA,
        ],
        compiler_params=pltpu.CompilerParams(
            use_tc_tiling_on_sc=False, needs_layout_passes=False),
    )(_edge_kernel_body)
    return k(ft, edge_index)


def _fin_body(u_ref, f_ref, o_ref):
    u = u_ref[0] + u_ref[1]
    den_pieces = []
    for h in range(H):
        d = u[:, ROW + h:ROW + h + 1]
        den_pieces.append(jnp.broadcast_to(d, (d.shape[0], 16)))
    den = jnp.concatenate(den_pieces, axis=1)
    num = u[:, :ROW]
    scaled = jnp.where(den > 0.0, num / jnp.where(den > 0.0, den, 1.0), 0.0)
    f = f_ref[...]
    att = scaled[:, 0:16]
    res = f[:, 0:16]
    for h in range(1, H):
        att = att + scaled[:, 16 * h:16 * (h + 1)]
        res = res + f[:, 16 * h:16 * (h + 1)]
    o_ref[...] = att * (2.0 / H) + res * (1.0 / H)


def _finalize(u, feat):
    blk = 2000
    return pl.pallas_call(
        _fin_body,
        grid=(N // blk,),
        in_specs=[pl.BlockSpec((NC, blk, AROW), lambda i: (0, i, 0)),
                  pl.BlockSpec((blk, D_IN), lambda i: (i, 0))],
        out_specs=pl.BlockSpec((blk, D), lambda i: (i, 0)),
        out_shape=jax.ShapeDtypeStruct((N, D), jnp.float32),
    )(u, feat)


def kernel(feat, edge_index, W):
    ft = _matmul(feat, W)
    ei_flat = edge_index.T.reshape(2 * E)
    u = _edge_pass(ft, ei_flat)
    return _finalize(u, feat)


# speedup vs baseline: 1.6214x; 1.6214x over previous
import functools

import jax
import jax.numpy as jnp
from jax import lax
from jax.experimental import pallas as pl
from jax.experimental.pallas import tpu as pltpu
from jax.experimental.pallas import tpu_sc as plsc

N = 10000
E = 320000
D_IN = 128
H = 8
D = 16
ROW = H * D
AROW = ROW + 16

NC = 2
NS = 16
NW = NC * NS
EPW = E // NW
CHUNK = 40
NCHUNK = EPW // CHUNK
NPAD = 10240
RPT = NPAD // NS
WCH = 40
SCBYTES = CHUNK * AROW * 4
assert EPW % CHUNK == 0 and NCHUNK % 2 == 0 and RPT % WCH == 0


def _mm_body(f_ref, w_ref, o_ref):
    o_ref[...] = (jnp.dot(f_ref[...], w_ref[...],
                          preferred_element_type=jnp.float32)
                  * 0.5).astype(jnp.bfloat16)


def _matmul(feat, W):
    blk = 2000
    return pl.pallas_call(
        _mm_body,
        grid=(N // blk,),
        in_specs=[pl.BlockSpec((blk, D_IN), lambda i: (i, 0)),
                  pl.BlockSpec((D_IN, ROW), lambda i: (0, 0))],
        out_specs=pl.BlockSpec((blk, ROW), lambda i: (i, 0)),
        out_shape=jax.ShapeDtypeStruct((N, ROW), jnp.bfloat16),
    )(feat, W)


def _edge_kernel_body(ft_hbm, ei_hbm, out_hbm,
                      src_i0, src_i1, dst_i0, dst_i1, sdst0, sdst1,
                      srows0, srows1, drows0, drows1, arows0, arows1,
                      acc_sh, sem_g0, sem_g1, sem_s0, sem_s1,
                      sem_i0, sem_i1):
    cid = lax.axis_index("c")
    sid = lax.axis_index("s")
    wid = cid * NS + sid
    ebase = wid * EPW

    src_i = (src_i0, src_i1)
    dst_i = (dst_i0, dst_i1)
    sdst = (sdst0, sdst1)
    srows = (srows0, srows1)
    drows = (drows0, drows1)
    arows = (arows0, arows1)
    sem_g = (sem_g0, sem_g1)
    sem_s = (sem_s0, sem_s1)
    sem_i = (sem_i0, sem_i1)

    for r in range(WCH):
        for c in range(AROW // 16):
            arows0[r, pl.ds(16 * c, 16)] = jnp.zeros((16,), jnp.float32)

    def _zero_body(i, _):
        pltpu.sync_copy(arows0, acc_sh.at[pl.ds(sid * RPT + i * WCH, WCH)])
        return 0
    lax.fori_loop(0, RPT // WCH, _zero_body, 0)
    plsc.subcore_barrier()

    one_hot = [(lax.iota(jnp.int32, 16) == h).astype(jnp.float32)
               for h in range(H)]

    pltpu.sync_copy(ei_hbm.at[0, pl.ds(ebase, CHUNK)], src_i0)
    pltpu.sync_copy(ei_hbm.at[1, pl.ds(ebase, CHUNK)], dst_i0)
    pltpu.async_copy(ft_hbm.at[src_i0], srows0, sem_g0)
    pltpu.async_copy(ft_hbm.at[dst_i0], drows0, sem_g0)
    pltpu.async_copy(ei_hbm.at[0, pl.ds(ebase + CHUNK, CHUNK)], src_i1,
                     sem_i1)
    pltpu.async_copy(ei_hbm.at[1, pl.ds(ebase + CHUNK, CHUNK)], dst_i1,
                     sem_i1)

    def _half_body(j, b):
        o = 1 - b
        pltpu.make_async_copy(ei_hbm.at[0, pl.ds(ebase, CHUNK)],
                              src_i[o], sem_i[o]).wait()
        pltpu.make_async_copy(ei_hbm.at[1, pl.ds(ebase, CHUNK)],
                              dst_i[o], sem_i[o]).wait()
        pltpu.async_copy(ft_hbm.at[src_i[o]], srows[o], sem_g[o])
        pltpu.async_copy(ft_hbm.at[dst_i[o]], drows[o], sem_g[o])
        pltpu.make_async_copy(ft_hbm.at[src_i[b]], srows[b], sem_g[b]).wait()
        pltpu.make_async_copy(ft_hbm.at[dst_i[b]], drows[b], sem_g[b]).wait()

        @pl.when(j >= 2)
        def _wait_prev_scatter():
            pltpu.make_async_copy(
                arows[b], acc_sh.at[sdst[b]], sem_s[b]).wait()

        for k in (0, 16, CHUNK - 16):
            sdst[b][pl.ds(k, 16)] = dst_i[b][pl.ds(k, 16)]
        nbase = ebase + jnp.minimum(j + 2, NCHUNK - 1) * CHUNK
        pltpu.async_copy(ei_hbm.at[0, pl.ds(nbase, CHUNK)], src_i[b],
                         sem_i[b])
        pltpu.async_copy(ei_hbm.at[1, pl.ds(nbase, CHUNK)], dst_i[b],
                         sem_i[b])

        @plsc.parallel_loop(0, CHUNK, 1, unroll=2)
        def _edge_body(e):
            sv, dv = [], []
            for g in range(H // 2):
                sp = plsc.unpack(
                    srows[b][e, pl.ds(32 * g, 32)],
                    format=plsc.PackFormat.INTERLEAVED,
                    preferred_element_type=jnp.float32)
                dp = plsc.unpack(
                    drows[b][e, pl.ds(32 * g, 32)],
                    format=plsc.PackFormat.INTERLEAVED,
                    preferred_element_type=jnp.float32)
                sv += [sp[0], sp[1]]
                dv += [dp[0], dp[1]]
            cs = [plsc.cumsum(sv[h] * dv[h]) for h in range(H)]
            m = [one_hot[h] * cs[h][15] for h in range(H)]
            a_vec = ((m[0] + m[1]) + (m[2] + m[3])) + \
                    ((m[4] + m[5]) + (m[6] + m[7]))
            ex = jnp.exp(a_vec)
            arows[b][e, pl.ds(ROW, 16)] = ex
            for h in range(H):
                arows[b][e, pl.ds(16 * h, 16)] = sv[h] * ex[h]

        pltpu.async_copy(arows[b], acc_sh.at[sdst[b]], sem_s[b], add=True)

    def _pair_body(jj, _):
        _half_body(jj * 2, 0)
        _half_body(jj * 2 + 1, 1)
        return 0
    lax.fori_loop(0, NCHUNK // 2, _pair_body, 0)

    pltpu.make_async_copy(arows0, acc_sh.at[sdst0], sem_s0).wait()
    pltpu.make_async_copy(arows1, acc_sh.at[sdst1], sem_s1).wait()
    pltpu.make_async_copy(ft_hbm.at[src_i0], srows0, sem_g0).wait()
    pltpu.make_async_copy(ft_hbm.at[dst_i0], drows0, sem_g0).wait()
    pltpu.make_async_copy(ei_hbm.at[0, pl.ds(ebase, CHUNK)],
                          src_i1, sem_i1).wait()
    pltpu.make_async_copy(ei_hbm.at[1, pl.ds(ebase, CHUNK)],
                          dst_i1, sem_i1).wait()
    plsc.subcore_barrier()

    def _out_body(i, _):
        rbase = sid * RPT + i * WCH
        pltpu.sync_copy(acc_sh.at[pl.ds(rbase, WCH)], arows0)
        pltpu.sync_copy(arows0, out_hbm.at[cid, pl.ds(rbase, WCH)])
        return 0
    lax.fori_loop(0, RPT // WCH, _out_body, 0)


def _edge_pass(ft, edge_index):
    mesh = plsc.VectorSubcoreMesh(core_axis_name="c", subcore_axis_name="s")
    k = functools.partial(
        pl.kernel,
        mesh=mesh,
        out_type=jax.ShapeDtypeStruct((NC, NPAD, AROW), jnp.float32),
        scratch_types=[
            pltpu.VMEM((CHUNK,), jnp.int32),
            pltpu.VMEM((CHUNK,), jnp.int32),
            pltpu.VMEM((CHUNK,), jnp.int32),
            pltpu.VMEM((CHUNK,), jnp.int32),
            pltpu.VMEM((CHUNK,), jnp.int32),
            pltpu.VMEM((CHUNK,), jnp.int32),
            pltpu.VMEM((CHUNK, ROW), jnp.bfloat16),
            pltpu.VMEM((CHUNK, ROW), jnp.bfloat16),
            pltpu.VMEM((CHUNK, ROW), jnp.bfloat16),
            pltpu.VMEM((CHUNK, ROW), jnp.bfloat16),
            pltpu.VMEM((CHUNK, AROW), jnp.float32),
            pltpu.VMEM((CHUNK, AROW), jnp.float32),
            pltpu.VMEM_SHARED((NPAD, AROW), jnp.float32),
            pltpu.SemaphoreType.DMA,
            pltpu.SemaphoreType.DMA,
            pltpu.SemaphoreType.DMA,
            pltpu.SemaphoreType.DMA,
            pltpu.SemaphoreType.DMA,
            pltpu.SemaphoreType.DMA,
        ],
        compiler_params=pltpu.CompilerParams(
            use_tc_tiling_on_sc=False, needs_layout_passes=False),
    )(_edge_kernel_body)
    return k(ft, edge_index)


def _fin_body(u_ref, f_ref, o_ref):
    u = u_ref[0] + u_ref[1]
    den_pieces = []
    for h in range(H):
        d = u[:, ROW + h:ROW + h + 1]
        den_pieces.append(jnp.broadcast_to(d, (d.shape[0], 16)))
    den = jnp.concatenate(den_pieces, axis=1)
    num = u[:, :ROW]
    scaled = jnp.where(den > 0.0, num / jnp.where(den > 0.0, den, 1.0), 0.0)
    f = f_ref[...]
    att = scaled[:, 0:16]
    res = f[:, 0:16]
    for h in range(1, H):
        att = att + scaled[:, 16 * h:16 * (h + 1)]
        res = res + f[:, 16 * h:16 * (h + 1)]
    o_ref[...] = att * (2.0 / H) + res * (1.0 / H)


def _finalize(u, feat):
    blk = 2000
    return pl.pallas_call(
        _fin_body,
        grid=(N // blk,),
        in_specs=[pl.BlockSpec((NC, blk, AROW), lambda i: (0, i, 0)),
                  pl.BlockSpec((blk, D_IN), lambda i: (i, 0))],
        out_specs=pl.BlockSpec((blk, D), lambda i: (i, 0)),
        out_shape=jax.ShapeDtypeStruct((N, D), jnp.float32),
    )(u, feat)


_PERM = [0] * ROW
for _g in range(H // 2):
    for _k in range(16):
        _PERM[32 * _g + 2 * _k] = 32 * _g + _k
        _PERM[32 * _g + 2 * _k + 1] = 32 * _g + 16 + _k


def kernel(feat, edge_index, W):
    ft = _matmul(feat, W[:, jnp.array(_PERM)])
    u = _edge_pass(ft, edge_index)
    return _finalize(u, feat)
